# Initial kernel scaffold; baseline (speedup 1.0000x reference)
import jax
import jax.numpy as jnp
from jax.experimental import pallas as pl


def kernel(token_ids, weight):
    raise NotImplementedError("write your pallas kernel here")


# speedup vs baseline: 4.9464x; 4.9464x over previous
import functools

import jax
import jax.numpy as jnp
from jax import lax
from jax.experimental import pallas as pl
from jax.experimental.pallas import tpu as pltpu
from jax.experimental.pallas import tpu_sc as plsc

_LANES = 128
_K = 16


def _emb_lookup(weight, idx_rows):
    R = idx_rows.shape[0]
    _, D = weight.shape
    info = plsc.get_sparse_core_info()
    num_cores = info.num_cores
    nw = num_cores * info.num_subcores
    rows_per_w = R // nw
    groups = rows_per_w // _K

    mesh = plsc.VectorSubcoreMesh(core_axis_name="c", subcore_axis_name="s")

    @functools.partial(
        pl.kernel,
        mesh=mesh,
        compiler_params=pltpu.CompilerParams(use_tc_tiling_on_sc=False),
        out_type=jax.ShapeDtypeStruct((R, _LANES, D), jnp.float32),
        scratch_types=[
            pltpu.VMEM((_K, _LANES), jnp.int32),
            pltpu.VMEM((_K, _LANES, D), jnp.float32),
            pltpu.SemaphoreType.DMA,
        ],
    )
    def emb(w_hbm, idx_hbm, out_hbm, idx_v, rows_v, sem):
        wid = lax.axis_index("s") * num_cores + lax.axis_index("c")
        base = wid * rows_per_w

        def body(g, carry):
            off = base + g * _K
            pltpu.sync_copy(idx_hbm.at[pl.ds(off, _K)], idx_v)
            copies = [
                pltpu.async_copy(w_hbm.at[idx_v.at[j]], rows_v.at[j], sem)
                for j in range(_K)
            ]
            for c in copies:
                c.wait()
            pltpu.sync_copy(rows_v, out_hbm.at[pl.ds(off, _K)])
            return carry

        lax.fori_loop(0, groups, body, 0)

    return emb(weight, idx_rows)


def kernel(token_ids, weight):
    b, h = token_ids.shape
    _, d = weight.shape
    n = b * h
    idx_rows = token_ids.reshape(n // _LANES, _LANES).astype(jnp.int32)
    out = _emb_lookup(weight, idx_rows)
    return out.reshape(b, h, d)


# speedup vs baseline: 5.0436x; 1.0196x over previous
import functools

import jax
import jax.numpy as jnp
from jax import lax
from jax.experimental import pallas as pl
from jax.experimental.pallas import tpu as pltpu
from jax.experimental.pallas import tpu_sc as plsc

_LANES = 128
_K = 10


def _emb_lookup(weight, idx_rows):
    R = idx_rows.shape[0]
    _, D = weight.shape
    info = plsc.get_sparse_core_info()
    num_cores = info.num_cores
    nw = num_cores * info.num_subcores
    rows_per_w = R // nw
    groups = rows_per_w // _K
    assert groups % 2 == 0
    npairs = groups // 2
    gsz = _K * _LANES

    mesh = plsc.VectorSubcoreMesh(core_axis_name="c", subcore_axis_name="s")

    @functools.partial(
        pl.kernel,
        mesh=mesh,
        compiler_params=pltpu.CompilerParams(use_tc_tiling_on_sc=False),
        out_type=jax.ShapeDtypeStruct((R * _LANES, D), jnp.float32),
        scratch_types=[
            pltpu.VMEM((_K, _LANES), jnp.int32),
            pltpu.VMEM((_K, _LANES), jnp.int32),
            pltpu.VMEM((gsz, D), jnp.float32),
            pltpu.VMEM((gsz, D), jnp.float32),
            pltpu.SemaphoreType.DMA,
            pltpu.SemaphoreType.DMA,
            pltpu.SemaphoreType.DMA,
            pltpu.SemaphoreType.DMA,
        ],
    )
    def emb(w_hbm, idx_hbm, out_hbm, idx0, idx1, rows0, rows1,
            gsem0, gsem1, ssem0, ssem1):
        idx_v = (idx0, idx1)
        rows_v = (rows0, rows1)
        gsem = (gsem0, gsem1)
        ssem = (ssem0, ssem1)

        wid = lax.axis_index("s") * num_cores + lax.axis_index("c")
        base = wid * rows_per_w

        def fire(g, ib):
            pltpu.sync_copy(idx_hbm.at[pl.ds(base + g * _K, _K)], idx_v[ib])
            for j in range(_K):
                pltpu.async_copy(
                    w_hbm.at[idx_v[ib].at[j]],
                    rows_v[ib].at[pl.ds(j * _LANES, _LANES)],
                    gsem[ib],
                )

        def wait_gathers(ib):
            for j in range(_K):
                pltpu.make_async_copy(
                    w_hbm.at[idx_v[ib].at[j]],
                    rows_v[ib].at[pl.ds(j * _LANES, _LANES)],
                    gsem[ib],
                ).wait()

        def store(g, ib):
            pltpu.async_copy(
                rows_v[ib], out_hbm.at[pl.ds((base + g * _K) * _LANES, gsz)],
                ssem[ib],
            )

        def wait_store(ib):
            pltpu.make_async_copy(
                rows_v[ib], out_hbm.at[pl.ds(base * _LANES, gsz)], ssem[ib]
            ).wait()

        fire(0, 0)
        fire(1, 1)

        def body(p, carry):
            g = 2 * p
            wait_gathers(0)
            store(g, 0)
            wait_gathers(1)
            store(g + 1, 1)

            @pl.when(p + 1 < npairs)
            def _():
                wait_store(0)
                fire(g + 2, 0)
                wait_store(1)
                fire(g + 3, 1)

            return carry

        lax.fori_loop(0, npairs, body, 0)
        wait_store(0)
        wait_store(1)

    return emb(weight, idx_rows)


def kernel(token_ids, weight):
    b, h = token_ids.shape
    _, d = weight.shape
    n = b * h
    idx_rows = token_ids.reshape(n // _LANES, _LANES).astype(jnp.int32)
    out = _emb_lookup(weight, idx_rows)
    return out.reshape(b, h, d)


# speedup vs baseline: 5.0499x; 1.0012x over previous
import functools

import jax
import jax.numpy as jnp
from jax import lax
from jax.experimental import pallas as pl
from jax.experimental.pallas import tpu as pltpu
from jax.experimental.pallas import tpu_sc as plsc

_LANES = 128
_K = 5
_NBUF = 4


def _emb_lookup(weight, idx_rows):
    R = idx_rows.shape[0]
    _, D = weight.shape
    info = plsc.get_sparse_core_info()
    num_cores = info.num_cores
    nw = num_cores * info.num_subcores
    rows_per_w = R // nw
    groups = rows_per_w // _K
    assert groups % _NBUF == 0
    rounds = groups // _NBUF
    gsz = _K * _LANES

    mesh = plsc.VectorSubcoreMesh(core_axis_name="c", subcore_axis_name="s")

    @functools.partial(
        pl.kernel,
        mesh=mesh,
        compiler_params=pltpu.CompilerParams(use_tc_tiling_on_sc=False),
        out_type=jax.ShapeDtypeStruct((R * _LANES, D), jnp.float32),
        scratch_types=[
            [pltpu.VMEM((_K, _LANES), jnp.int32) for _ in range(_NBUF)],
            [pltpu.VMEM((gsz, D), jnp.float32) for _ in range(_NBUF)],
            [pltpu.SemaphoreType.DMA for _ in range(_NBUF)],
            [pltpu.SemaphoreType.DMA for _ in range(_NBUF)],
        ],
    )
    def emb(w_hbm, idx_hbm, out_hbm, idx_v, rows_v, gsem, ssem):
        wid = lax.axis_index("s") * num_cores + lax.axis_index("c")
        base = wid * rows_per_w

        def fire(g, ib):
            pltpu.sync_copy(idx_hbm.at[pl.ds(base + g * _K, _K)], idx_v[ib])
            for j in range(_K):
                pltpu.async_copy(
                    w_hbm.at[idx_v[ib].at[j]],
                    rows_v[ib].at[pl.ds(j * _LANES, _LANES)],
                    gsem[ib],
                )

        def wait_gathers(ib):
            for j in range(_K):
                pltpu.make_async_copy(
                    w_hbm.at[idx_v[ib].at[j]],
                    rows_v[ib].at[pl.ds(j * _LANES, _LANES)],
                    gsem[ib],
                ).wait()

        def store(g, ib):
            pltpu.async_copy(
                rows_v[ib], out_hbm.at[pl.ds((base + g * _K) * _LANES, gsz)],
                ssem[ib],
            )

        def wait_store(ib):
            pltpu.make_async_copy(
                rows_v[ib], out_hbm.at[pl.ds(base * _LANES, gsz)], ssem[ib]
            ).wait()

        for b in range(_NBUF - 1):
            fire(b, b)

        def body(r, carry):
            for b in range(_NBUF):
                g = r * _NBUF + b
                wait_gathers(b)
                store(g, b)
                bprev = (b - 1) % _NBUF
                gf = g + _NBUF - 1

                @pl.when(gf < groups)
                def _():
                    if b == 0:
                        @pl.when(r > 0)
                        def _():
                            wait_store(bprev)
                    else:
                        wait_store(bprev)
                    fire(gf, bprev)

            return carry

        lax.fori_loop(0, rounds, body, 0)
        for b in range(_NBUF):
            wait_store(b)

    return emb(weight, idx_rows)


def kernel(token_ids, weight):
    b, h = token_ids.shape
    _, d = weight.shape
    n = b * h
    idx_rows = token_ids.reshape(n // _LANES, _LANES).astype(jnp.int32)
    out = _emb_lookup(weight, idx_rows)
    return out.reshape(b, h, d)
